# Initial kernel scaffold; baseline (speedup 1.0000x reference)
import jax
import jax.numpy as jnp
from jax.experimental import pallas as pl


def kernel(x, table):
    raise NotImplementedError("write your pallas kernel here")


# speedup vs baseline: 3.2520x; 3.2520x over previous
import jax
import jax.numpy as jnp
from jax.experimental import pallas as pl


HIDDEN = 1024
ROW_BLOCK = 1024


def _add_kernel(x_ref, t_ref, o_ref):
    o_ref[...] = x_ref[...] + t_ref[...]


def kernel(x, table):
    b, n, h = x.shape
    num_blocks = n // ROW_BLOCK

    grid = (num_blocks, b)
    out = pl.pallas_call(
        _add_kernel,
        grid=grid,
        in_specs=[
            pl.BlockSpec((1, ROW_BLOCK, h), lambda i, j: (j, i, 0)),
            pl.BlockSpec((ROW_BLOCK, h), lambda i, j: (i, 0)),
        ],
        out_specs=pl.BlockSpec((1, ROW_BLOCK, h), lambda i, j: (j, i, 0)),
        out_shape=jax.ShapeDtypeStruct((b, n, h), x.dtype),
    )(x, table)
    return out


# speedup vs baseline: 3.4414x; 1.0582x over previous
import jax
import jax.numpy as jnp
from jax.experimental import pallas as pl


HIDDEN = 1024
ROW_BLOCK = 2048


def _add_kernel(x_ref, t_ref, o_ref):
    o_ref[...] = x_ref[...] + t_ref[...]


def kernel(x, table):
    b, n, h = x.shape
    num_blocks = n // ROW_BLOCK

    grid = (num_blocks, b)
    out = pl.pallas_call(
        _add_kernel,
        grid=grid,
        in_specs=[
            pl.BlockSpec((1, ROW_BLOCK, h), lambda i, j: (j, i, 0)),
            pl.BlockSpec((ROW_BLOCK, h), lambda i, j: (i, 0)),
        ],
        out_specs=pl.BlockSpec((1, ROW_BLOCK, h), lambda i, j: (j, i, 0)),
        out_shape=jax.ShapeDtypeStruct((b, n, h), x.dtype),
    )(x, table)
    return out
